# baseline (speedup 1.0000x reference)
import functools

import jax
import jax.numpy as jnp
from jax import lax
from jax.experimental import pallas as pl
from jax.experimental.pallas import tpu as pltpu
from jax.experimental.pallas import tpu_sc as plsc

_BLK = 2000
_INT_MAX = 2147483647


def _tc_dist_block(data_ref, w_ref, bp_ref, min_ref, arg_ref):
    i = pl.program_id(0)
    x = data_ref[...]
    e = jnp.dot(x, w_ref[...], preferred_element_type=jnp.float32)
    diff = e + bp_ref[...]
    d2 = jnp.sum(diff * diff, axis=1, keepdims=True)
    m = jnp.min(d2)
    ridx = lax.broadcasted_iota(jnp.int32, d2.shape, 0)
    a = jnp.min(jnp.where(d2 == m, ridx, _INT_MAX))
    min_ref[0, 0, 0] = m
    arg_ref[0, 0, 0] = i * _BLK + a


def _perm16(x, perm):
    dn = lax.GatherDimensionNumbers(
        offset_dims=(), collapsed_slice_dims=(0,), start_index_map=(0,))
    return lax.gather(x, perm[:, None], dn, (1,),
                      mode=lax.GatherScatterMode.PROMISE_IN_BOUNDS)


def _make_sc_reduce_gather(n_rows, feat_dim, nb_pad):
    mesh = plsc.VectorSubcoreMesh(core_axis_name="c", subcore_axis_name="s")

    @functools.partial(
        pl.kernel,
        mesh=mesh,
        out_type=[
            jax.ShapeDtypeStruct((1, feat_dim), jnp.float32),
            jax.ShapeDtypeStruct((16,), jnp.int32),
        ],
        scratch_types=[
            pltpu.VMEM((nb_pad,), jnp.float32),
            pltpu.VMEM((nb_pad,), jnp.int32),
            pltpu.VMEM((1, feat_dim), jnp.float32),
            pltpu.VMEM((16,), jnp.int32),
            pltpu.VMEM((16,), jnp.int32),
        ],
    )
    def sc_fn(bm_hbm, ba_hbm, data_hbm, lab_hbm, row_out, lab_out,
              bm_v, ba_v, row_v, lab_v, lsel_v):
        wid = lax.axis_index("s") * 2 + lax.axis_index("c")

        @pl.when(wid == 0)
        def _():
            pltpu.sync_copy(bm_hbm, bm_v)
            pltpu.sync_copy(ba_hbm, ba_v)
            bestv = bm_v[pl.ds(0, 16)]
            besti = ba_v[pl.ds(0, 16)]
            for j in range(1, nb_pad // 16):
                v = bm_v[pl.ds(j * 16, 16)]
                a = ba_v[pl.ds(j * 16, 16)]
                better = (v < bestv) | ((v == bestv) & (a < besti))
                bestv = jnp.where(better, v, bestv)
                besti = jnp.where(better, a, besti)
            lanes = lax.iota(jnp.int32, 16)
            for s in (8, 4, 2, 1):
                v = _perm16(bestv, lanes ^ s)
                a = _perm16(besti, lanes ^ s)
                better = (v < bestv) | ((v == bestv) & (a < besti))
                bestv = jnp.where(better, v, bestv)
                besti = jnp.where(better, a, besti)
            idx = besti[0]
            pltpu.sync_copy(data_hbm.at[pl.ds(idx, 1)], row_v)
            pltpu.sync_copy(row_v, row_out)
            base = jnp.minimum((idx // 8) * 8, jnp.int32(n_rows - 16))
            pltpu.sync_copy(lab_hbm.at[pl.ds(base, 16)], lab_v)
            sel = jnp.where(lanes == idx - base, lab_v[pl.ds(0, 16)], 0)
            for s in (8, 4, 2, 1):
                sel = sel + _perm16(sel, lanes ^ s)
            lsel_v[...] = sel
            pltpu.sync_copy(lsel_v, lab_out)

    return sc_fn


def kernel(prototype_vector, data, label, W, b):
    n, feat = data.shape
    latent = W.shape[1]
    nb = n // _BLK

    bp = (b - prototype_vector).reshape(1, latent)

    blockmin, blockarg = pl.pallas_call(
        _tc_dist_block,
        grid=(nb,),
        in_specs=[
            pl.BlockSpec((_BLK, feat), lambda i: (i, 0)),
            pl.BlockSpec((feat, latent), lambda i: (0, 0)),
            pl.BlockSpec((1, latent), lambda i: (0, 0)),
        ],
        out_specs=[
            pl.BlockSpec((1, 1, 1), lambda i: (i, 0, 0), memory_space=pltpu.SMEM),
            pl.BlockSpec((1, 1, 1), lambda i: (i, 0, 0), memory_space=pltpu.SMEM),
        ],
        out_shape=[
            jax.ShapeDtypeStruct((nb, 1, 1), jnp.float32),
            jax.ShapeDtypeStruct((nb, 1, 1), jnp.int32),
        ],
    )(data, W, bp)

    nb_pad = ((nb + 15) // 16) * 16
    bm = jnp.pad(blockmin.reshape(nb), (0, nb_pad - nb),
                 constant_values=jnp.inf)
    ba = jnp.pad(blockarg.reshape(nb), (0, nb_pad - nb),
                 constant_values=_INT_MAX)

    row, lab16 = _make_sc_reduce_gather(n, feat, nb_pad)(bm, ba, data, label)
    return (row.reshape(feat), lab16[0])
